# baseline (speedup 1.0000x reference)
import functools

import jax
import jax.numpy as jnp
from jax import lax
from jax.experimental import pallas as pl
from jax.experimental.pallas import tpu as pltpu
from jax.experimental.pallas import tpu_sc as plsc

_NC = 2
_NS = 16
_NW = _NC * _NS

_IB = 128
_PW = 128
_L = 16
_NBUF = 5


@functools.lru_cache(maxsize=None)
def _build(hist, batch, vocab, d):
  mesh = plsc.VectorSubcoreMesh(
      core_axis_name="c", subcore_axis_name="s",
      num_cores=_NC, num_subcores=_NS)

  @functools.partial(
      pl.kernel,
      out_type=jax.ShapeDtypeStruct((hist, d, batch), jnp.float32),
      mesh=mesh,
      scratch_types=[
          pltpu.VMEM((hist, _IB), jnp.int32),
          pltpu.VMEM((_NBUF, _IB, _PW), jnp.float32),
          pltpu.VMEM((2, d, _IB), jnp.float32),
          pltpu.SemaphoreType.DMA,
          pltpu.SemaphoreType.DMA,
      ],
      compiler_params=pltpu.CompilerParams(needs_layout_passes=False),
  )
  def lookup(idx_hbm, table_hbm, out_hbm, idx_v, rows_v, outt_v, gsem, osem):
    wid = lax.axis_index("s") * _NC + lax.axis_index("c")
    b0 = wid * _IB

    pltpu.sync_copy(idx_hbm.at[:, pl.ds(b0, _IB)], idx_v)

    def fire(h, j):
      pltpu.make_async_copy(
          table_hbm.at[idx_v.at[h]], rows_v.at[j], gsem).start()

    def store_desc(h, j):
      return pltpu.make_async_copy(
          outt_v.at[j % 2], out_hbm.at[h, :, pl.ds(b0, _IB)], osem)

    lanes = lax.iota(jnp.int32, _L)
    cvecs = [lanes + (g * _L) for g in range(d // _L)]

    for j in range(_NBUF - 1):
      fire(j, j)

    def body(i, carry):
      for j in range(2 * _NBUF):
        h = i * (2 * _NBUF) + j
        pltpu.make_async_copy(
            table_hbm.at[idx_v.at[h]], rows_v.at[j % _NBUF], gsem).wait()

        @pl.when(h + _NBUF - 1 < hist)
        def _():
          fire(h + _NBUF - 1, (j + _NBUF - 1) % _NBUF)

        rows2 = rows_v.at[j % _NBUF]
        outt2 = outt_v.at[j % 2]

        @pl.when(h >= 2)
        def _():
          store_desc(h, j).wait()

        @plsc.parallel_loop(0, _IB, step=1, unroll=8)
        def _(b):
          bsplat = jnp.full((_L,), b, jnp.int32)
          for g in range(d // _L):
            plsc.store_scatter(
                outt2, [cvecs[g], bsplat], rows2[b, pl.ds(g * _L, _L)])

        store_desc(h, j).start()
      return carry

    lax.fori_loop(0, hist // (2 * _NBUF), body, 0)
    store_desc(0, 0).wait()
    store_desc(0, 1).wait()

  return lookup


def kernel(inputs, embeddings):
  b, h = inputs.shape
  vocab, d = embeddings.shape
  idx_t = jnp.transpose(inputs.astype(jnp.int32))
  tpad = jnp.pad(embeddings, ((0, 0), (0, _PW - d)))
  out = _build(h, b, vocab, d)(idx_t, tpad)
  return jnp.transpose(out, (2, 0, 1))


# speedup vs baseline: 1.5470x; 1.5470x over previous
import functools

import jax
import jax.numpy as jnp
from jax import lax
from jax.experimental import pallas as pl
from jax.experimental.pallas import tpu as pltpu
from jax.experimental.pallas import tpu_sc as plsc

_NC = 2
_NS = 16
_NW = _NC * _NS

_IB = 128
_PW = 128
_L = 16
_NBUF = 5


@functools.lru_cache(maxsize=None)
def _build(hist, batch, vocab, d):
  mesh = plsc.VectorSubcoreMesh(
      core_axis_name="c", subcore_axis_name="s",
      num_cores=_NC, num_subcores=_NS)

  @functools.partial(
      pl.kernel,
      out_type=jax.ShapeDtypeStruct((hist, d, batch), jnp.float32),
      mesh=mesh,
      scratch_types=[
          pltpu.VMEM((hist, _IB), jnp.int32),
          pltpu.VMEM((_NBUF, _IB, _PW), jnp.float32),
          pltpu.VMEM((2, d, _IB), jnp.float32),
          pltpu.SemaphoreType.DMA,
          pltpu.SemaphoreType.DMA,
      ],
      compiler_params=pltpu.CompilerParams(needs_layout_passes=False),
  )
  def lookup(idx_hbm, table_hbm, out_hbm, idx_v, rows_v, outt_v, gsem, osem):
    wid = lax.axis_index("s") * _NC + lax.axis_index("c")
    b0 = wid * _IB

    pltpu.sync_copy(idx_hbm.at[:, pl.ds(b0, _IB)], idx_v)

    def fire(h, j):
      pltpu.make_async_copy(
          table_hbm.at[idx_v.at[h]], rows_v.at[j], gsem).start()

    def store_desc(h, j):
      return pltpu.make_async_copy(
          outt_v.at[j % 2], out_hbm.at[h, :, pl.ds(b0, _IB)], osem)

    lanes = lax.iota(jnp.int32, _L)
    rot = [(lanes + dd) % _L for dd in range(_L)]

    for j in range(_NBUF - 1):
      fire(j, j)

    def body(h, carry):
      sr = lax.rem(h, _NBUF)
      so = lax.rem(h, 2)
      pltpu.make_async_copy(
          table_hbm.at[idx_v.at[h]], rows_v.at[sr], gsem).wait()

      @pl.when(h + _NBUF - 1 < hist)
      def _():
        fire(h + _NBUF - 1, lax.rem(h + _NBUF - 1, _NBUF))

      rows2 = rows_v.at[sr]
      outt2 = outt_v.at[so]

      @pl.when(h >= 2)
      def _():
        store_desc(h, so).wait()

      @plsc.parallel_loop(0, _IB, step=_L)
      def _(b0):
        bvec = jnp.full((_L,), b0, jnp.int32) + lanes
        for c0 in range(0, d, _L):
          for dd in range(_L):
            cvec = rot[dd] + c0
            v = plsc.load_gather(rows2, [bvec, cvec])
            plsc.store_scatter(outt2, [cvec, bvec], v)

      store_desc(h, so).start()
      return carry

    lax.fori_loop(0, hist, body, 0)
    store_desc(0, 0).wait()
    store_desc(0, 1).wait()

  return lookup


def kernel(inputs, embeddings):
  b, h = inputs.shape
  vocab, d = embeddings.shape
  idx_t = jnp.transpose(inputs.astype(jnp.int32))
  tpad = jnp.pad(embeddings, ((0, 0), (0, _PW - d)))
  out = _build(h, b, vocab, d)(idx_t, tpad)
  return jnp.transpose(out, (2, 0, 1))
